# baseline (speedup 1.0000x reference)
import jax
import jax.numpy as jnp
from jax import lax
from jax.experimental import pallas as pl
from jax.experimental.pallas import tpu as pltpu
from jax.experimental.pallas import tpu_sc as plsc

B = 16384
L = 20
D = 32
DD = 2 * D
NC = 2
NS = 16
NW = NC * NS
BPW = B // NW
CH = 128
NCHUNK = BPW // CH
GSZ = 128
NG_I = BPW // GSZ

ID_V = 100000
TI_V = 50000
TCH = 1568
ID_PAD = ((ID_V + NW * TCH - 1) // (NW * TCH)) * NW * TCH // NW
TI_PAD = ((TI_V + NW * TCH - 1) // (NW * TCH)) * NW * TCH // NW


def _transpose_table(tabT_hbm, out_hbm, v_rows, rows_per_w, wid,
                     stage_v, outst_v, sem_i, sem_o, iota32):
    nch = rows_per_w // TCH
    w_r0 = wid * rows_per_w
    out_desc = None

    for i in range(nch):
        r0 = jnp.minimum(w_r0 + i * TCH, v_rows - TCH)
        pltpu.async_copy(
            tabT_hbm.at[:, pl.ds(r0, TCH)], stage_v, sem_i
        ).wait()
        if out_desc is not None:
            out_desc.wait()

        def blk_body(bi, _):
            br = bi * 16
            rv32 = iota32 + br * D
            for f in range(D):
                v = stage_v[f, pl.ds(br, 16)]
                plsc.store_scatter(outst_v, [rv32 + f], v)
            return 0

        lax.fori_loop(0, TCH // 16, blk_body, 0)

        out_desc = pltpu.async_copy(
            outst_v, out_hbm.at[pl.ds(r0 * D, TCH * D)], sem_o
        )

    out_desc.wait()


def _relayout_body(idtabT_hbm, titabT_hbm, idlin_hbm, titlin_hbm,
                   stage_v, outst_v, sem_i, sem_o):
    wid = lax.axis_index("s") * NC + lax.axis_index("c")
    iota32 = lax.iota(jnp.int32, 16) * D
    _transpose_table(titabT_hbm, titlin_hbm, TI_V, TI_PAD, wid,
                     stage_v, outst_v, sem_i, sem_o, iota32)
    _transpose_table(idtabT_hbm, idlin_hbm, ID_V, ID_PAD, wid,
                     stage_v, outst_v, sem_i, sem_o, iota32)


def _gather_body(ids_hbm, toksT_hbm, idtab_hbm, titab_hbm, out_hbm,
                 tokT_v, ids_v, idrows_v, sum_v, out_v, row0_v, nz_v, inv_v,
                 sem_in, sem_id, sem_g0, sem_g1, sem_o0, sem_o1):
    wid = lax.axis_index("s") * NC + lax.axis_index("c")
    base = wid * BPW

    in_descs = [
        pltpu.async_copy(toksT_hbm.at[l, pl.ds(base, BPW)], tokT_v.at[l], sem_in)
        for l in range(L)
    ]
    pltpu.sync_copy(ids_hbm.at[pl.ds(base, BPW)], ids_v)
    pltpu.sync_copy(titab_hbm.at[pl.ds(0, 1), :], row0_v)
    for d in in_descs:
        d.wait()

    id_descs = [
        pltpu.async_copy(
            idtab_hbm.at[ids_v.at[pl.ds(j * GSZ, GSZ)]],
            idrows_v.at[pl.ds(j * GSZ, GSZ), :],
            sem_id,
        )
        for j in range(NG_I)
    ]

    sems_g = (sem_g0, sem_g1)
    sems_o = (sem_o0, sem_o1)
    zero16 = jnp.zeros((16,), jnp.float32)
    g_descs = [None] * NCHUNK
    o_descs = [None] * NCHUNK

    row0a = row0_v[0, pl.ds(0, 16)]
    row0b = row0_v[0, pl.ds(16, 16)]

    def prep_chunk(c):
        buf = c % 2

        def zero_body(r, _):
            sum_v[buf, r, pl.ds(0, 16)] = zero16
            sum_v[buf, r, pl.ds(16, 16)] = zero16
            return 0

        lax.fori_loop(0, CH, zero_body, 0)

        def group_body(g, _):
            rs = c * CH + g * 16
            nz = jnp.zeros((16,), jnp.float32)
            for l in range(L):
                t = tokT_v[l, pl.ds(rs, 16)]
                nz = nz + jnp.where(t == 0, 1.0, 0.0)
            denom = jnp.maximum(jnp.float32(L) - nz, 1.0)
            nz_v[buf, pl.ds(g * 16, 16)] = nz
            inv_v[buf, pl.ds(g * 16, 16)] = 1.0 / denom
            return 0

        lax.fori_loop(0, CH // 16, group_body, 0)

        return [
            pltpu.async_copy(
                titab_hbm.at[tokT_v.at[l, pl.ds(c * CH, CH)]],
                sum_v.at[buf],
                sems_g[buf],
                add=True,
            )
            for l in range(L)
        ]

    g_descs[0] = prep_chunk(0)

    for c in range(NCHUNK):
        buf = c % 2
        if c + 1 < NCHUNK:
            g_descs[c + 1] = prep_chunk(c + 1)
        for d in g_descs[c]:
            d.wait()
        if c == 0:
            for d in id_descs:
                d.wait()
        if c >= 2:
            o_descs[c - 2].wait()

        def row_body(r, _):
            s0 = sum_v[buf, r, pl.ds(0, 16)]
            s1 = sum_v[buf, r, pl.ds(16, 16)]
            nzr = nz_v[buf, pl.ds(r, 16)][0]
            invr = inv_v[buf, pl.ds(r, 16)][0]
            out_v[buf, r, pl.ds(0, 16)] = idrows_v[c * CH + r, pl.ds(0, 16)]
            out_v[buf, r, pl.ds(16, 16)] = idrows_v[c * CH + r, pl.ds(16, 16)]
            out_v[buf, r, pl.ds(32, 16)] = (s0 - nzr * row0a) * invr
            out_v[buf, r, pl.ds(48, 16)] = (s1 - nzr * row0b) * invr
            return 0

        lax.fori_loop(0, CH, row_body, 0)

        o_descs[c] = pltpu.async_copy(
            out_v.at[buf],
            out_hbm.at[pl.ds(base + c * CH, CH), :],
            sems_o[buf],
        )

    o_descs[NCHUNK - 2].wait()
    o_descs[NCHUNK - 1].wait()


@jax.jit
def kernel(movie_id, movie_title_tokens, id_embedding_table, title_embedding_table):
    toksT = movie_title_tokens.T
    idtabT = id_embedding_table.T
    titabT = title_embedding_table.T

    mesh = plsc.VectorSubcoreMesh(core_axis_name="c", subcore_axis_name="s")
    params = pltpu.CompilerParams(
        needs_layout_passes=False, use_tc_tiling_on_sc=False
    )

    relayout = pl.kernel(
        _relayout_body,
        out_type=(
            jax.ShapeDtypeStruct((NW * ID_PAD * D,), jnp.float32),
            jax.ShapeDtypeStruct((NW * TI_PAD * D,), jnp.float32),
        ),
        mesh=mesh,
        compiler_params=params,
        scratch_types=[
            pltpu.VMEM((D, TCH), jnp.float32),
            pltpu.VMEM((TCH * D,), jnp.float32),
            pltpu.SemaphoreType.DMA,
            pltpu.SemaphoreType.DMA,
        ],
    )
    idlin, titlin = relayout(idtabT, titabT)
    idlin = idlin.reshape(NW * ID_PAD, D)
    titlin = titlin.reshape(NW * TI_PAD, D)

    run = pl.kernel(
        _gather_body,
        out_type=jax.ShapeDtypeStruct((B, DD), jnp.float32),
        mesh=mesh,
        compiler_params=params,
        scratch_types=[
            pltpu.VMEM((L, BPW), jnp.int32),
            pltpu.VMEM((BPW,), jnp.int32),
            pltpu.VMEM((BPW, D), jnp.float32),
            pltpu.VMEM((2, CH, D), jnp.float32),
            pltpu.VMEM((2, CH, DD), jnp.float32),
            pltpu.VMEM((1, D), jnp.float32),
            pltpu.VMEM((2, CH + 16), jnp.float32),
            pltpu.VMEM((2, CH + 16), jnp.float32),
            pltpu.SemaphoreType.DMA,
            pltpu.SemaphoreType.DMA,
            pltpu.SemaphoreType.DMA,
            pltpu.SemaphoreType.DMA,
            pltpu.SemaphoreType.DMA,
            pltpu.SemaphoreType.DMA,
        ],
    )
    return run(movie_id, toksT, idlin, titlin)


# speedup vs baseline: 1.4247x; 1.4247x over previous
import jax
import jax.numpy as jnp
from jax import lax
from jax.experimental import pallas as pl
from jax.experimental.pallas import tpu as pltpu
from jax.experimental.pallas import tpu_sc as plsc

B = 16384
L = 20
D = 32
DD = 2 * D
NC = 2
NS = 16
NW = NC * NS
BPW = B // NW
CH = 128
NCHUNK = BPW // CH
GSZ = 128
NG_I = BPW // GSZ

ID_V = 100000
TI_V = 50000
TCH = 224
ID_PAD = ((ID_V + NW * TCH - 1) // (NW * TCH)) * NW * TCH // NW
TI_PAD = ((TI_V + NW * TCH - 1) // (NW * TCH)) * NW * TCH // NW


def _transpose_table(tabT_hbm, out_hbm, v_rows, rows_per_w, wid,
                     stages, outst_v, sem_i, sem_o, iota, fvs):
    nch = rows_per_w // TCH
    w_r0 = wid * rows_per_w

    def chunk_r0(i):
        return jnp.minimum(w_r0 + i * TCH, v_rows - TCH)

    in_descs = [None, None]
    out_descs = [None, None]
    in_descs[0] = pltpu.async_copy(
        tabT_hbm.at[:, pl.ds(chunk_r0(0), TCH)], stages[0], sem_i
    )

    for i in range(nch):
        buf = i % 2
        if i + 1 < nch:
            in_descs[1 - buf] = pltpu.async_copy(
                tabT_hbm.at[:, pl.ds(chunk_r0(i + 1), TCH)],
                stages[1 - buf], sem_i,
            )
        in_descs[buf].wait()
        if out_descs[buf] is not None:
            out_descs[buf].wait()
        stage_v = stages[buf]
        ost = outst_v.at[buf]

        def blk_body(bi, _):
            br = bi * 16
            rb = br + iota
            sb0 = rb * D
            sb1 = sb0 + 16
            for c in range(16):
                fv = fvs[c]
                v0 = plsc.load_gather(stage_v, [fv, rb])
                plsc.store_scatter(ost, [sb0 + fv], v0)
                v1 = plsc.load_gather(stage_v, [fv + 16, rb])
                plsc.store_scatter(ost, [sb1 + fv], v1)
            return 0

        lax.fori_loop(0, TCH // 16, blk_body, 0)

        out_descs[buf] = pltpu.async_copy(
            ost, out_hbm.at[pl.ds(chunk_r0(i) * D, TCH * D)], sem_o
        )

    for d in out_descs:
        if d is not None:
            d.wait()


def _relayout_body(idtabT_hbm, titabT_hbm, idlin_hbm, titlin_hbm,
                   stage0_v, stage1_v, outst_v, sem_i, sem_o):
    wid = lax.axis_index("s") * NC + lax.axis_index("c")
    iota = lax.iota(jnp.int32, 16)
    fvs = [(iota + c) & 15 for c in range(16)]
    stages = (stage0_v, stage1_v)
    _transpose_table(titabT_hbm, titlin_hbm, TI_V, TI_PAD, wid,
                     stages, outst_v, sem_i, sem_o, iota, fvs)
    _transpose_table(idtabT_hbm, idlin_hbm, ID_V, ID_PAD, wid,
                     stages, outst_v, sem_i, sem_o, iota, fvs)


def _gather_body(ids_hbm, toksT_hbm, idtab_hbm, titab_hbm, out_hbm,
                 tokT_v, ids_v, idrows_v, sum_v, out_v, row0_v, nz_v, inv_v,
                 sem_in, sem_id, sem_g0, sem_g1, sem_o0, sem_o1):
    wid = lax.axis_index("s") * NC + lax.axis_index("c")
    base = wid * BPW

    in_descs = [
        pltpu.async_copy(toksT_hbm.at[l, pl.ds(base, BPW)], tokT_v.at[l], sem_in)
        for l in range(L)
    ]
    pltpu.sync_copy(ids_hbm.at[pl.ds(base, BPW)], ids_v)
    pltpu.sync_copy(titab_hbm.at[pl.ds(0, 1), :], row0_v)
    for d in in_descs:
        d.wait()

    id_descs = [
        pltpu.async_copy(
            idtab_hbm.at[ids_v.at[pl.ds(j * GSZ, GSZ)]],
            idrows_v.at[pl.ds(j * GSZ, GSZ), :],
            sem_id,
        )
        for j in range(NG_I)
    ]

    sems_g = (sem_g0, sem_g1)
    sems_o = (sem_o0, sem_o1)
    zero16 = jnp.zeros((16,), jnp.float32)
    g_descs = [None] * NCHUNK
    o_descs = [None] * NCHUNK

    row0a = row0_v[0, pl.ds(0, 16)]
    row0b = row0_v[0, pl.ds(16, 16)]

    def prep_chunk(c):
        buf = c % 2

        def zero_body(r, _):
            sum_v[buf, r, pl.ds(0, 16)] = zero16
            sum_v[buf, r, pl.ds(16, 16)] = zero16
            return 0

        lax.fori_loop(0, CH, zero_body, 0)

        def group_body(g, _):
            rs = c * CH + g * 16
            nz = jnp.zeros((16,), jnp.float32)
            for l in range(L):
                t = tokT_v[l, pl.ds(rs, 16)]
                nz = nz + jnp.where(t == 0, 1.0, 0.0)
            denom = jnp.maximum(jnp.float32(L) - nz, 1.0)
            nz_v[buf, pl.ds(g * 16, 16)] = nz
            inv_v[buf, pl.ds(g * 16, 16)] = 1.0 / denom
            return 0

        lax.fori_loop(0, CH // 16, group_body, 0)

        return [
            pltpu.async_copy(
                titab_hbm.at[tokT_v.at[l, pl.ds(c * CH, CH)]],
                sum_v.at[buf],
                sems_g[buf],
                add=True,
            )
            for l in range(L)
        ]

    g_descs[0] = prep_chunk(0)

    for c in range(NCHUNK):
        buf = c % 2
        if c + 1 < NCHUNK:
            g_descs[c + 1] = prep_chunk(c + 1)
        for d in g_descs[c]:
            d.wait()
        if c == 0:
            for d in id_descs:
                d.wait()
        if c >= 2:
            o_descs[c - 2].wait()

        def row_body(r, _):
            s0 = sum_v[buf, r, pl.ds(0, 16)]
            s1 = sum_v[buf, r, pl.ds(16, 16)]
            nzr = nz_v[buf, pl.ds(r, 16)][0]
            invr = inv_v[buf, pl.ds(r, 16)][0]
            out_v[buf, r, pl.ds(0, 16)] = idrows_v[c * CH + r, pl.ds(0, 16)]
            out_v[buf, r, pl.ds(16, 16)] = idrows_v[c * CH + r, pl.ds(16, 16)]
            out_v[buf, r, pl.ds(32, 16)] = (s0 - nzr * row0a) * invr
            out_v[buf, r, pl.ds(48, 16)] = (s1 - nzr * row0b) * invr
            return 0

        lax.fori_loop(0, CH, row_body, 0)

        o_descs[c] = pltpu.async_copy(
            out_v.at[buf],
            out_hbm.at[pl.ds(base + c * CH, CH), :],
            sems_o[buf],
        )

    o_descs[NCHUNK - 2].wait()
    o_descs[NCHUNK - 1].wait()


@jax.jit
def kernel(movie_id, movie_title_tokens, id_embedding_table, title_embedding_table):
    toksT = movie_title_tokens.T
    idtabT = id_embedding_table.T
    titabT = title_embedding_table.T

    mesh = plsc.VectorSubcoreMesh(core_axis_name="c", subcore_axis_name="s")
    params = pltpu.CompilerParams(
        needs_layout_passes=False, use_tc_tiling_on_sc=False
    )

    relayout = pl.kernel(
        _relayout_body,
        out_type=(
            jax.ShapeDtypeStruct((NW * ID_PAD * D,), jnp.float32),
            jax.ShapeDtypeStruct((NW * TI_PAD * D,), jnp.float32),
        ),
        mesh=mesh,
        compiler_params=params,
        scratch_types=[
            pltpu.VMEM((D, TCH), jnp.float32),
            pltpu.VMEM((D, TCH), jnp.float32),
            pltpu.VMEM((2, TCH * D), jnp.float32),
            pltpu.SemaphoreType.DMA,
            pltpu.SemaphoreType.DMA,
        ],
    )
    idlin, titlin = relayout(idtabT, titabT)
    idlin = idlin.reshape(NW * ID_PAD, D)
    titlin = titlin.reshape(NW * TI_PAD, D)

    run = pl.kernel(
        _gather_body,
        out_type=jax.ShapeDtypeStruct((B, DD), jnp.float32),
        mesh=mesh,
        compiler_params=params,
        scratch_types=[
            pltpu.VMEM((L, BPW), jnp.int32),
            pltpu.VMEM((BPW,), jnp.int32),
            pltpu.VMEM((BPW, D), jnp.float32),
            pltpu.VMEM((2, CH, D), jnp.float32),
            pltpu.VMEM((2, CH, DD), jnp.float32),
            pltpu.VMEM((1, D), jnp.float32),
            pltpu.VMEM((2, CH + 16), jnp.float32),
            pltpu.VMEM((2, CH + 16), jnp.float32),
            pltpu.SemaphoreType.DMA,
            pltpu.SemaphoreType.DMA,
            pltpu.SemaphoreType.DMA,
            pltpu.SemaphoreType.DMA,
            pltpu.SemaphoreType.DMA,
            pltpu.SemaphoreType.DMA,
        ],
    )
    return run(movie_id, toksT, idlin, titlin)


# speedup vs baseline: 1.4404x; 1.0110x over previous
import jax
import jax.numpy as jnp
from jax import lax
from jax.experimental import pallas as pl
from jax.experimental.pallas import tpu as pltpu
from jax.experimental.pallas import tpu_sc as plsc

B = 16384
L = 20
D = 32
DD = 2 * D
NC = 2
NS = 16
NW = NC * NS
BPW = B // NW
CH = 128
NCHUNK = BPW // CH
GSZ = 128
NG_I = BPW // GSZ

ID_V = 100000
TI_V = 50000
TCH = 448
ID_PAD = ((ID_V + NW * TCH - 1) // (NW * TCH)) * NW * TCH // NW
TI_PAD = ((TI_V + NW * TCH - 1) // (NW * TCH)) * NW * TCH // NW


def _transpose_table(tabT_hbm, out_hbm, v_rows, rows_per_w, wid,
                     stages, outst_v, sem_i, sem_o, iota, fvs):
    nch = rows_per_w // TCH
    w_r0 = wid * rows_per_w

    def chunk_r0(i):
        return jnp.minimum(w_r0 + i * TCH, v_rows - TCH)

    in_descs = [None, None]
    out_descs = [None, None]
    in_descs[0] = pltpu.async_copy(
        tabT_hbm.at[:, pl.ds(chunk_r0(0), TCH)], stages[0], sem_i
    )

    for i in range(nch):
        buf = i % 2
        if i + 1 < nch:
            in_descs[1 - buf] = pltpu.async_copy(
                tabT_hbm.at[:, pl.ds(chunk_r0(i + 1), TCH)],
                stages[1 - buf], sem_i,
            )
        in_descs[buf].wait()
        if out_descs[buf] is not None:
            out_descs[buf].wait()
        stage_v = stages[buf]
        ost = outst_v.at[buf]

        def blk_body(bi, _):
            br = bi * 16
            rb = br + iota
            sb0 = rb * D
            sb1 = sb0 + 16
            for c in range(16):
                fv = fvs[c]
                v0 = plsc.load_gather(stage_v, [fv, rb])
                plsc.store_scatter(ost, [sb0 + fv], v0)
                v1 = plsc.load_gather(stage_v, [fv + 16, rb])
                plsc.store_scatter(ost, [sb1 + fv], v1)
            return 0

        lax.fori_loop(0, TCH // 16, blk_body, 0)

        out_descs[buf] = pltpu.async_copy(
            ost, out_hbm.at[pl.ds(chunk_r0(i) * D, TCH * D)], sem_o
        )

    for d in out_descs:
        if d is not None:
            d.wait()


def _relayout_body(idtabT_hbm, titabT_hbm, idlin_hbm, titlin_hbm,
                   stage0_v, stage1_v, outst_v, sem_i, sem_o):
    wid = lax.axis_index("s") * NC + lax.axis_index("c")
    iota = lax.iota(jnp.int32, 16)
    fvs = [(iota + c) & 15 for c in range(16)]
    stages = (stage0_v, stage1_v)
    _transpose_table(titabT_hbm, titlin_hbm, TI_V, TI_PAD, wid,
                     stages, outst_v, sem_i, sem_o, iota, fvs)
    _transpose_table(idtabT_hbm, idlin_hbm, ID_V, ID_PAD, wid,
                     stages, outst_v, sem_i, sem_o, iota, fvs)


def _gather_body(ids_hbm, toksT_hbm, idtab_hbm, titab_hbm, out_hbm,
                 tokT_v, ids_v, idrows_v, sum_v, out_v, row0_v, nz_v, inv_v,
                 sem_in, sem_id, sem_g0, sem_g1, sem_o0, sem_o1):
    wid = lax.axis_index("s") * NC + lax.axis_index("c")
    base = wid * BPW

    in_descs = [
        pltpu.async_copy(toksT_hbm.at[l, pl.ds(base, BPW)], tokT_v.at[l], sem_in)
        for l in range(L)
    ]
    pltpu.sync_copy(ids_hbm.at[pl.ds(base, BPW)], ids_v)
    pltpu.sync_copy(titab_hbm.at[pl.ds(0, 1), :], row0_v)
    for d in in_descs:
        d.wait()

    id_descs = [
        pltpu.async_copy(
            idtab_hbm.at[ids_v.at[pl.ds(j * GSZ, GSZ)]],
            idrows_v.at[pl.ds(j * GSZ, GSZ), :],
            sem_id,
        )
        for j in range(NG_I)
    ]

    sems_g = (sem_g0, sem_g1)
    sems_o = (sem_o0, sem_o1)
    zero16 = jnp.zeros((16,), jnp.float32)
    g_descs = [None] * NCHUNK
    o_descs = [None] * NCHUNK

    row0a = row0_v[0, pl.ds(0, 16)]
    row0b = row0_v[0, pl.ds(16, 16)]

    def prep_chunk(c):
        buf = c % 2

        def zero_body(r, _):
            sum_v[buf, r, pl.ds(0, 16)] = zero16
            sum_v[buf, r, pl.ds(16, 16)] = zero16
            return 0

        lax.fori_loop(0, CH, zero_body, 0)

        def group_body(g, _):
            rs = c * CH + g * 16
            nz = jnp.zeros((16,), jnp.float32)
            for l in range(L):
                t = tokT_v[l, pl.ds(rs, 16)]
                nz = nz + jnp.where(t == 0, 1.0, 0.0)
            denom = jnp.maximum(jnp.float32(L) - nz, 1.0)
            nz_v[buf, pl.ds(g * 16, 16)] = nz
            inv_v[buf, pl.ds(g * 16, 16)] = 1.0 / denom
            return 0

        lax.fori_loop(0, CH // 16, group_body, 0)

        return [
            pltpu.async_copy(
                titab_hbm.at[tokT_v.at[l, pl.ds(c * CH, CH)]],
                sum_v.at[buf],
                sems_g[buf],
                add=True,
            )
            for l in range(L)
        ]

    g_descs[0] = prep_chunk(0)

    for c in range(NCHUNK):
        buf = c % 2
        if c + 1 < NCHUNK:
            g_descs[c + 1] = prep_chunk(c + 1)
        for d in g_descs[c]:
            d.wait()
        if c == 0:
            for d in id_descs:
                d.wait()
        if c >= 2:
            o_descs[c - 2].wait()

        def row_body(r, _):
            s0 = sum_v[buf, r, pl.ds(0, 16)]
            s1 = sum_v[buf, r, pl.ds(16, 16)]
            nzr = nz_v[buf, pl.ds(r, 16)][0]
            invr = inv_v[buf, pl.ds(r, 16)][0]
            out_v[buf, r, pl.ds(0, 16)] = idrows_v[c * CH + r, pl.ds(0, 16)]
            out_v[buf, r, pl.ds(16, 16)] = idrows_v[c * CH + r, pl.ds(16, 16)]
            out_v[buf, r, pl.ds(32, 16)] = (s0 - nzr * row0a) * invr
            out_v[buf, r, pl.ds(48, 16)] = (s1 - nzr * row0b) * invr
            return 0

        lax.fori_loop(0, CH, row_body, 0)

        o_descs[c] = pltpu.async_copy(
            out_v.at[buf],
            out_hbm.at[pl.ds(base + c * CH, CH), :],
            sems_o[buf],
        )

    o_descs[NCHUNK - 2].wait()
    o_descs[NCHUNK - 1].wait()


@jax.jit
def kernel(movie_id, movie_title_tokens, id_embedding_table, title_embedding_table):
    toksT = movie_title_tokens.T
    idtabT = id_embedding_table.T
    titabT = title_embedding_table.T

    mesh = plsc.VectorSubcoreMesh(core_axis_name="c", subcore_axis_name="s")
    params = pltpu.CompilerParams(
        needs_layout_passes=False, use_tc_tiling_on_sc=False
    )

    relayout = pl.kernel(
        _relayout_body,
        out_type=(
            jax.ShapeDtypeStruct((NW * ID_PAD * D,), jnp.float32),
            jax.ShapeDtypeStruct((NW * TI_PAD * D,), jnp.float32),
        ),
        mesh=mesh,
        compiler_params=params,
        scratch_types=[
            pltpu.VMEM((D, TCH), jnp.float32),
            pltpu.VMEM((D, TCH), jnp.float32),
            pltpu.VMEM((2, TCH * D), jnp.float32),
            pltpu.SemaphoreType.DMA,
            pltpu.SemaphoreType.DMA,
        ],
    )
    idlin, titlin = relayout(idtabT, titabT)
    idlin = idlin.reshape(NW * ID_PAD, D)
    titlin = titlin.reshape(NW * TI_PAD, D)

    run = pl.kernel(
        _gather_body,
        out_type=jax.ShapeDtypeStruct((B, DD), jnp.float32),
        mesh=mesh,
        compiler_params=params,
        scratch_types=[
            pltpu.VMEM((L, BPW), jnp.int32),
            pltpu.VMEM((BPW,), jnp.int32),
            pltpu.VMEM((BPW, D), jnp.float32),
            pltpu.VMEM((2, CH, D), jnp.float32),
            pltpu.VMEM((2, CH, DD), jnp.float32),
            pltpu.VMEM((1, D), jnp.float32),
            pltpu.VMEM((2, CH + 16), jnp.float32),
            pltpu.VMEM((2, CH + 16), jnp.float32),
            pltpu.SemaphoreType.DMA,
            pltpu.SemaphoreType.DMA,
            pltpu.SemaphoreType.DMA,
            pltpu.SemaphoreType.DMA,
            pltpu.SemaphoreType.DMA,
            pltpu.SemaphoreType.DMA,
        ],
    )
    return run(movie_id, toksT, idlin, titlin)


# speedup vs baseline: 1.7320x; 1.2024x over previous
import jax
import jax.numpy as jnp
from jax import lax
from jax.experimental import pallas as pl
from jax.experimental.pallas import tpu as pltpu
from jax.experimental.pallas import tpu_sc as plsc

B = 16384
L = 20
D = 32
DD = 2 * D
NC = 2
NS = 16
NW = NC * NS
BPW = B // NW
CH = 128
NCHUNK = BPW // CH
GSZ = 128
NG_I = BPW // GSZ

ID_V = 100000
TI_V = 50000
TCH = 448
ID_PAD = ((ID_V + NW * TCH - 1) // (NW * TCH)) * NW * TCH // NW
TI_PAD = ((TI_V + NW * TCH - 1) // (NW * TCH)) * NW * TCH // NW


def _transpose_table(tabT_hbm, out_hbm, v_rows, rows_per_w, wid,
                     stages, outst_v, sem_i, sem_o, iota, fvs):
    nch = rows_per_w // TCH
    w_r0 = wid * rows_per_w

    def chunk_r0(i):
        return jnp.minimum(w_r0 + i * TCH, v_rows - TCH)

    in_descs = [None, None]
    out_descs = [None, None]
    in_descs[0] = pltpu.async_copy(
        tabT_hbm.at[:, pl.ds(chunk_r0(0), TCH)], stages[0], sem_i
    )

    for i in range(nch):
        buf = i % 2
        if i + 1 < nch:
            in_descs[1 - buf] = pltpu.async_copy(
                tabT_hbm.at[:, pl.ds(chunk_r0(i + 1), TCH)],
                stages[1 - buf], sem_i,
            )
        in_descs[buf].wait()
        if out_descs[buf] is not None:
            out_descs[buf].wait()
        stage_v = stages[buf]
        ost = outst_v.at[buf]

        def blk_body(bi, _):
            br = bi * 16
            rb = br + iota
            sb0 = rb * D
            sb1 = sb0 + 16
            for c in range(16):
                fv = fvs[c]
                v0 = plsc.load_gather(stage_v, [fv, rb])
                plsc.store_scatter(ost, [sb0 + fv], v0)
                v1 = plsc.load_gather(stage_v, [fv + 16, rb])
                plsc.store_scatter(ost, [sb1 + fv], v1)
            return 0

        lax.fori_loop(0, TCH // 16, blk_body, 0)

        out_descs[buf] = pltpu.async_copy(
            ost, out_hbm.at[pl.ds(chunk_r0(i) * D, TCH * D)], sem_o
        )

    for d in out_descs:
        if d is not None:
            d.wait()


def _relayout_body(titabT_hbm, titlin_hbm,
                   stage0_v, stage1_v, outst_v, sem_i, sem_o):
    wid = lax.axis_index("s") * NC + lax.axis_index("c")
    iota = lax.iota(jnp.int32, 16)
    fvs = [(iota + c) & 15 for c in range(16)]
    stages = (stage0_v, stage1_v)
    _transpose_table(titabT_hbm, titlin_hbm, TI_V, TI_PAD, wid,
                     stages, outst_v, sem_i, sem_o, iota, fvs)


def _gather_body(ids_hbm, toksT_hbm, idtabT_hbm, titab_hbm, out_hbm,
                 tokT_v, ids_v, idcols_v, idrows_v, sum_v, out_v, row0_v,
                 nz_v, inv_v,
                 sem_in, sem_id, sem_g0, sem_g1, sem_o0, sem_o1):
    wid = lax.axis_index("s") * NC + lax.axis_index("c")
    base = wid * BPW
    iota = lax.iota(jnp.int32, 16)

    in_descs = [
        pltpu.async_copy(toksT_hbm.at[l, pl.ds(base, BPW)], tokT_v.at[l], sem_in)
        for l in range(L)
    ]
    pltpu.sync_copy(ids_hbm.at[pl.ds(base, BPW)], ids_v)
    pltpu.sync_copy(titab_hbm.at[pl.ds(0, 1), :], row0_v)
    for d in in_descs:
        d.wait()

    id_descs = [
        pltpu.async_copy(
            idtabT_hbm.at[f].at[ids_v.at[pl.ds(j * GSZ, GSZ)]],
            idcols_v.at[f, pl.ds(j * GSZ, GSZ)],
            sem_id,
        )
        for f in range(D)
        for j in range(NG_I)
    ]

    sems_g = (sem_g0, sem_g1)
    sems_o = (sem_o0, sem_o1)
    zero16 = jnp.zeros((16,), jnp.float32)
    g_descs = [None] * NCHUNK
    o_descs = [None] * NCHUNK

    row0a = row0_v[0, pl.ds(0, 16)]
    row0b = row0_v[0, pl.ds(16, 16)]

    def prep_chunk(c):
        buf = c % 2

        def zero_body(r, _):
            sum_v[buf, r, pl.ds(0, 16)] = zero16
            sum_v[buf, r, pl.ds(16, 16)] = zero16
            return 0

        lax.fori_loop(0, CH, zero_body, 0)

        def group_body(g, _):
            rs = c * CH + g * 16
            nz = jnp.zeros((16,), jnp.float32)
            for l in range(L):
                t = tokT_v[l, pl.ds(rs, 16)]
                nz = nz + jnp.where(t == 0, 1.0, 0.0)
            denom = jnp.maximum(jnp.float32(L) - nz, 1.0)
            nz_v[buf, pl.ds(g * 16, 16)] = nz
            inv_v[buf, pl.ds(g * 16, 16)] = 1.0 / denom
            return 0

        lax.fori_loop(0, CH // 16, group_body, 0)

        return [
            pltpu.async_copy(
                titab_hbm.at[tokT_v.at[l, pl.ds(c * CH, CH)]],
                sum_v.at[buf],
                sems_g[buf],
                add=True,
            )
            for l in range(L)
        ]

    g_descs[0] = prep_chunk(0)

    for c in range(NCHUNK):
        buf = c % 2
        if c + 1 < NCHUNK:
            g_descs[c + 1] = prep_chunk(c + 1)
        for d in g_descs[c]:
            d.wait()
        if c == 0:
            for d in id_descs:
                d.wait()

            def id_blk(bi, _):
                rb = bi * 16 + iota
                for cc in range(16):
                    fv = (iota + cc) & 15
                    v0 = plsc.load_gather(idcols_v, [fv, rb])
                    plsc.store_scatter(idrows_v, [rb, fv], v0)
                    v1 = plsc.load_gather(idcols_v, [fv + 16, rb])
                    plsc.store_scatter(idrows_v, [rb, fv + 16], v1)
                return 0

            lax.fori_loop(0, BPW // 16, id_blk, 0)
        if c >= 2:
            o_descs[c - 2].wait()

        def row_body(r, _):
            s0 = sum_v[buf, r, pl.ds(0, 16)]
            s1 = sum_v[buf, r, pl.ds(16, 16)]
            nzr = nz_v[buf, pl.ds(r, 16)][0]
            invr = inv_v[buf, pl.ds(r, 16)][0]
            out_v[buf, r, pl.ds(0, 16)] = idrows_v[c * CH + r, pl.ds(0, 16)]
            out_v[buf, r, pl.ds(16, 16)] = idrows_v[c * CH + r, pl.ds(16, 16)]
            out_v[buf, r, pl.ds(32, 16)] = (s0 - nzr * row0a) * invr
            out_v[buf, r, pl.ds(48, 16)] = (s1 - nzr * row0b) * invr
            return 0

        lax.fori_loop(0, CH, row_body, 0)

        o_descs[c] = pltpu.async_copy(
            out_v.at[buf],
            out_hbm.at[pl.ds(base + c * CH, CH), :],
            sems_o[buf],
        )

    o_descs[NCHUNK - 2].wait()
    o_descs[NCHUNK - 1].wait()


@jax.jit
def kernel(movie_id, movie_title_tokens, id_embedding_table, title_embedding_table):
    toksT = movie_title_tokens.T
    idtabT = id_embedding_table.T
    titabT = title_embedding_table.T

    mesh = plsc.VectorSubcoreMesh(core_axis_name="c", subcore_axis_name="s")
    params = pltpu.CompilerParams(
        needs_layout_passes=False, use_tc_tiling_on_sc=False
    )

    relayout = pl.kernel(
        _relayout_body,
        out_type=jax.ShapeDtypeStruct((NW * TI_PAD * D,), jnp.float32),
        mesh=mesh,
        compiler_params=params,
        scratch_types=[
            pltpu.VMEM((D, TCH), jnp.float32),
            pltpu.VMEM((D, TCH), jnp.float32),
            pltpu.VMEM((2, TCH * D), jnp.float32),
            pltpu.SemaphoreType.DMA,
            pltpu.SemaphoreType.DMA,
        ],
    )
    titlin = relayout(titabT)
    titlin = titlin.reshape(NW * TI_PAD, D)

    run = pl.kernel(
        _gather_body,
        out_type=jax.ShapeDtypeStruct((B, DD), jnp.float32),
        mesh=mesh,
        compiler_params=params,
        scratch_types=[
            pltpu.VMEM((L, BPW), jnp.int32),
            pltpu.VMEM((BPW,), jnp.int32),
            pltpu.VMEM((D, BPW), jnp.float32),
            pltpu.VMEM((BPW, D), jnp.float32),
            pltpu.VMEM((2, CH, D), jnp.float32),
            pltpu.VMEM((2, CH, DD), jnp.float32),
            pltpu.VMEM((1, D), jnp.float32),
            pltpu.VMEM((2, CH + 16), jnp.float32),
            pltpu.VMEM((2, CH + 16), jnp.float32),
            pltpu.SemaphoreType.DMA,
            pltpu.SemaphoreType.DMA,
            pltpu.SemaphoreType.DMA,
            pltpu.SemaphoreType.DMA,
            pltpu.SemaphoreType.DMA,
            pltpu.SemaphoreType.DMA,
        ],
    )
    return run(movie_id, toksT, idtabT, titlin)


# speedup vs baseline: 1.7345x; 1.0014x over previous
import jax
import jax.numpy as jnp
from jax import lax
from jax.experimental import pallas as pl
from jax.experimental.pallas import tpu as pltpu
from jax.experimental.pallas import tpu_sc as plsc

B = 16384
L = 20
D = 32
DD = 2 * D
NC = 2
NS = 16
NW = NC * NS
BPW = B // NW
CH = 128
NCHUNK = BPW // CH
GSZ = 128
NG_I = BPW // GSZ

ID_V = 100000
TI_V = 50000
TCH = 448
ID_PAD = ((ID_V + NW * TCH - 1) // (NW * TCH)) * NW * TCH // NW
TI_PAD = ((TI_V + NW * TCH - 1) // (NW * TCH)) * NW * TCH // NW


def _transpose_table(tabT_hbm, out_hbm, v_rows, rows_per_w, wid,
                     stages, outst_v, sem_i, sem_o, iota, fvs):
    nch = rows_per_w // TCH
    w_r0 = wid * rows_per_w

    def chunk_r0(i):
        return jnp.minimum(w_r0 + i * TCH, v_rows - TCH)

    in_descs = [None, None]
    out_descs = [None, None]
    in_descs[0] = pltpu.async_copy(
        tabT_hbm.at[:, pl.ds(chunk_r0(0), TCH)], stages[0], sem_i
    )

    for i in range(nch):
        buf = i % 2
        if i + 1 < nch:
            in_descs[1 - buf] = pltpu.async_copy(
                tabT_hbm.at[:, pl.ds(chunk_r0(i + 1), TCH)],
                stages[1 - buf], sem_i,
            )
        in_descs[buf].wait()
        if out_descs[buf] is not None:
            out_descs[buf].wait()
        stage_v = stages[buf]
        ost = outst_v.at[buf]

        def blk_body(bi, _):
            br = bi * 16
            rb = br + iota
            sb0 = rb * D
            sb1 = sb0 + 16
            for c in range(16):
                fv = fvs[c]
                v0 = plsc.load_gather(stage_v, [fv, rb])
                plsc.store_scatter(ost, [sb0 + fv], v0)
                v1 = plsc.load_gather(stage_v, [fv + 16, rb])
                plsc.store_scatter(ost, [sb1 + fv], v1)
            return 0

        lax.fori_loop(0, TCH // 16, blk_body, 0)

        out_descs[buf] = pltpu.async_copy(
            ost, out_hbm.at[pl.ds(chunk_r0(i) * D, TCH * D)], sem_o
        )

    for d in out_descs:
        if d is not None:
            d.wait()


def _relayout_body(titabT_hbm, titlin_hbm,
                   stage0_v, stage1_v, outst_v, sem_i, sem_o):
    wid = lax.axis_index("s") * NC + lax.axis_index("c")
    iota = lax.iota(jnp.int32, 16)
    fvs = [(iota + c) & 15 for c in range(16)]
    stages = (stage0_v, stage1_v)
    _transpose_table(titabT_hbm, titlin_hbm, TI_V, TI_PAD, wid,
                     stages, outst_v, sem_i, sem_o, iota, fvs)


def _gather_body(ids_hbm, toksT_hbm, idtabT_hbm, titab_hbm, out_hbm,
                 tokT_v, ids_v, idcols_v, idrows_v, sum_v, out_v, row0_v,
                 nz_v, inv_v,
                 sem_in, sem_id, sem_g0, sem_g1, sem_o0, sem_o1):
    wid = lax.axis_index("s") * NC + lax.axis_index("c")
    base = wid * BPW
    iota = lax.iota(jnp.int32, 16)

    in_descs = [
        pltpu.async_copy(toksT_hbm.at[l, pl.ds(base, BPW)], tokT_v.at[l], sem_in)
        for l in range(L)
    ]
    pltpu.sync_copy(ids_hbm.at[pl.ds(base, BPW)], ids_v)
    pltpu.sync_copy(titab_hbm.at[pl.ds(0, 1), :], row0_v)

    id_descs = [
        pltpu.async_copy(
            idtabT_hbm.at[f].at[ids_v.at[pl.ds(j * GSZ, GSZ)]],
            idcols_v.at[f, pl.ds(j * GSZ, GSZ)],
            sem_id,
        )
        for f in range(D)
        for j in range(NG_I)
    ]
    for d in in_descs:
        d.wait()

    sems_g = (sem_g0, sem_g1)
    sems_o = (sem_o0, sem_o1)
    zero16 = jnp.zeros((16,), jnp.float32)
    g_descs = [None] * NCHUNK
    o_descs = [None] * NCHUNK

    row0a = row0_v[0, pl.ds(0, 16)]
    row0b = row0_v[0, pl.ds(16, 16)]

    def prep_chunk(c):
        buf = c % 2

        def zero_body(r, _):
            sum_v[buf, r, pl.ds(0, 16)] = zero16
            sum_v[buf, r, pl.ds(16, 16)] = zero16
            return 0

        lax.fori_loop(0, CH, zero_body, 0)

        def group_body(g, _):
            rs = c * CH + g * 16
            nz = jnp.zeros((16,), jnp.float32)
            for l in range(L):
                t = tokT_v[l, pl.ds(rs, 16)]
                nz = nz + jnp.where(t == 0, 1.0, 0.0)
            denom = jnp.maximum(jnp.float32(L) - nz, 1.0)
            nz_v[buf, pl.ds(g * 16, 16)] = nz
            inv_v[buf, pl.ds(g * 16, 16)] = 1.0 / denom
            return 0

        lax.fori_loop(0, CH // 16, group_body, 0)

        return [
            pltpu.async_copy(
                titab_hbm.at[tokT_v.at[l, pl.ds(c * CH, CH)]],
                sum_v.at[buf],
                sems_g[buf],
                add=True,
            )
            for l in range(L)
        ]

    g_descs[0] = prep_chunk(0)

    for c in range(NCHUNK):
        buf = c % 2
        if c + 1 < NCHUNK:
            g_descs[c + 1] = prep_chunk(c + 1)
        for d in g_descs[c]:
            d.wait()
        if c == 0:
            for d in id_descs:
                d.wait()

            def id_blk(bi, _):
                rb = bi * 16 + iota
                for cc in range(16):
                    fv = (iota + cc) & 15
                    v0 = plsc.load_gather(idcols_v, [fv, rb])
                    plsc.store_scatter(idrows_v, [rb, fv], v0)
                    v1 = plsc.load_gather(idcols_v, [fv + 16, rb])
                    plsc.store_scatter(idrows_v, [rb, fv + 16], v1)
                return 0

            lax.fori_loop(0, BPW // 16, id_blk, 0)
        if c >= 2:
            o_descs[c - 2].wait()

        def row_body(r, _):
            s0 = sum_v[buf, r, pl.ds(0, 16)]
            s1 = sum_v[buf, r, pl.ds(16, 16)]
            nzr = nz_v[buf, pl.ds(r, 16)][0]
            invr = inv_v[buf, pl.ds(r, 16)][0]
            out_v[buf, r, pl.ds(0, 16)] = idrows_v[c * CH + r, pl.ds(0, 16)]
            out_v[buf, r, pl.ds(16, 16)] = idrows_v[c * CH + r, pl.ds(16, 16)]
            out_v[buf, r, pl.ds(32, 16)] = (s0 - nzr * row0a) * invr
            out_v[buf, r, pl.ds(48, 16)] = (s1 - nzr * row0b) * invr
            return 0

        lax.fori_loop(0, CH, row_body, 0)

        o_descs[c] = pltpu.async_copy(
            out_v.at[buf],
            out_hbm.at[pl.ds(base + c * CH, CH), :],
            sems_o[buf],
        )

    o_descs[NCHUNK - 2].wait()
    o_descs[NCHUNK - 1].wait()


@jax.jit
def kernel(movie_id, movie_title_tokens, id_embedding_table, title_embedding_table):
    toksT = movie_title_tokens.T
    idtabT = id_embedding_table.T
    titabT = title_embedding_table.T

    mesh = plsc.VectorSubcoreMesh(core_axis_name="c", subcore_axis_name="s")
    params = pltpu.CompilerParams(
        needs_layout_passes=False, use_tc_tiling_on_sc=False
    )

    relayout = pl.kernel(
        _relayout_body,
        out_type=jax.ShapeDtypeStruct((NW * TI_PAD * D,), jnp.float32),
        mesh=mesh,
        compiler_params=params,
        scratch_types=[
            pltpu.VMEM((D, TCH), jnp.float32),
            pltpu.VMEM((D, TCH), jnp.float32),
            pltpu.VMEM((2, TCH * D), jnp.float32),
            pltpu.SemaphoreType.DMA,
            pltpu.SemaphoreType.DMA,
        ],
    )
    titlin = relayout(titabT)
    titlin = titlin.reshape(NW * TI_PAD, D)

    run = pl.kernel(
        _gather_body,
        out_type=jax.ShapeDtypeStruct((B, DD), jnp.float32),
        mesh=mesh,
        compiler_params=params,
        scratch_types=[
            pltpu.VMEM((L, BPW), jnp.int32),
            pltpu.VMEM((BPW,), jnp.int32),
            pltpu.VMEM((D, BPW), jnp.float32),
            pltpu.VMEM((BPW, D), jnp.float32),
            pltpu.VMEM((2, CH, D), jnp.float32),
            pltpu.VMEM((2, CH, DD), jnp.float32),
            pltpu.VMEM((1, D), jnp.float32),
            pltpu.VMEM((2, CH + 16), jnp.float32),
            pltpu.VMEM((2, CH + 16), jnp.float32),
            pltpu.SemaphoreType.DMA,
            pltpu.SemaphoreType.DMA,
            pltpu.SemaphoreType.DMA,
            pltpu.SemaphoreType.DMA,
            pltpu.SemaphoreType.DMA,
            pltpu.SemaphoreType.DMA,
        ],
    )
    return run(movie_id, toksT, idtabT, titlin)
